# Initial kernel scaffold; baseline (speedup 1.0000x reference)
import jax
import jax.numpy as jnp
from jax.experimental import pallas as pl


def kernel(embeds, centroids):
    raise NotImplementedError("write your pallas kernel here")


# speedup vs baseline: 75.4318x; 75.4318x over previous
import math

import jax
import jax.numpy as jnp
from jax.experimental import pallas as pl

_M_TILE = 1024


def _nn_dist_kernel(e_ref, c_ref, o_ref):
    e = e_ref[...]
    c = c_ref[...]
    enorm = jnp.sum(e * e, axis=1, keepdims=True)
    cnorm = jnp.sum(c * c, axis=1)[None, :]
    prod = jax.lax.dot_general(
        e, c,
        dimension_numbers=(((1,), (1,)), ((), ())),
        preferred_element_type=jnp.float32,
        precision=jax.lax.Precision.HIGHEST,
    )
    dist_sq = enorm + cnorm - 2.0 * prod
    dmin = jnp.min(dist_sq, axis=1)
    o_ref[...] = jnp.sqrt(dmin).reshape(o_ref.shape)


def kernel(embeds, centroids):
    B, N, D = embeds.shape
    P = centroids.shape[0]
    M = B * N
    e2 = embeds.reshape(M, D)
    n_tiles = M // _M_TILE
    rows_out = _M_TILE // 128

    out = pl.pallas_call(
        _nn_dist_kernel,
        grid=(n_tiles,),
        in_specs=[
            pl.BlockSpec((_M_TILE, D), lambda i: (i, 0)),
            pl.BlockSpec((P, D), lambda i: (0, 0)),
        ],
        out_specs=pl.BlockSpec((rows_out, 128), lambda i: (i, 0)),
        out_shape=jax.ShapeDtypeStruct((n_tiles * rows_out, 128), jnp.float32),
    )(e2, centroids)

    h = int(math.sqrt(N))
    score = out.reshape(B, 1, h, h)
    loss = jnp.zeros(())
    return (loss, score)


# speedup vs baseline: 220.2962x; 2.9205x over previous
import math

import jax
import jax.numpy as jnp
from jax.experimental import pallas as pl

_M_TILE = 1024


def _nn_dist_kernel(e_ref, c_ref, o_ref):
    e = e_ref[...]
    c = c_ref[...]
    enorm = jnp.sum(e * e, axis=1, keepdims=True)
    cnorm = jnp.sum(c * c, axis=1)[None, :]
    prod = jax.lax.dot_general(
        e.astype(jnp.bfloat16), c.astype(jnp.bfloat16),
        dimension_numbers=(((1,), (1,)), ((), ())),
        preferred_element_type=jnp.float32,
    )
    dist_sq = enorm + cnorm - 2.0 * prod
    dmin = jnp.min(dist_sq, axis=1)
    o_ref[...] = jnp.sqrt(dmin).reshape(o_ref.shape)


def kernel(embeds, centroids):
    B, N, D = embeds.shape
    P = centroids.shape[0]
    M = B * N
    e2 = embeds.reshape(M, D)
    n_tiles = M // _M_TILE
    rows_out = _M_TILE // 128

    out = pl.pallas_call(
        _nn_dist_kernel,
        grid=(n_tiles,),
        in_specs=[
            pl.BlockSpec((_M_TILE, D), lambda i: (i, 0)),
            pl.BlockSpec((P, D), lambda i: (0, 0)),
        ],
        out_specs=pl.BlockSpec((rows_out, 128), lambda i: (i, 0)),
        out_shape=jax.ShapeDtypeStruct((n_tiles * rows_out, 128), jnp.float32),
    )(e2, centroids)

    h = int(math.sqrt(N))
    score = out.reshape(B, 1, h, h)
    loss = jnp.zeros(())
    return (loss, score)
